# baseline (speedup 1.0000x reference)
import functools

import jax
import jax.numpy as jnp
from jax import lax
from jax.experimental import pallas as pl
from jax.experimental.pallas import tpu as pltpu
from jax.experimental.pallas import tpu_sc as plsc

BATCH = 16384
CTX_LEN = 20
OUT_LEN = 20
EMBED_DIM = 32

NC = 2
NS = 16
NW = NC * NS
LANES = 16

BPSB = 4
RB = BPSB * CTX_LEN
JBLK = (BATCH // NW) // BPSB
NBUF = 8
HALF = EMBED_DIM // 2
CW = BPSB * EMBED_DIM
PKW = EMBED_DIM // 2

_INV_CTX = 1.0 / CTX_LEN


def _shufxor(v, s):
    perm = lax.iota(jnp.int32, LANES) ^ s
    return lax.gather(
        v,
        perm[:, None],
        dimension_numbers=lax.GatherDimensionNumbers(
            offset_dims=(), collapsed_slice_dims=(0,), start_index_map=(0,)),
        slice_sizes=(1,),
        mode=lax.GatherScatterMode.PROMISE_IN_BOUNDS,
    )


def _reduce16(ps):
    cur = ps
    for s in (8, 4, 2, 1):
        n = len(cur) // 2
        m = (lax.iota(jnp.int32, LANES) & s) == 0
        nxt = []
        for i in range(n):
            a, b = cur[i], cur[i + n]
            nxt.append(jnp.where(m, a + _shufxor(a, s), b + _shufxor(b, s)))
        cur = nxt
    return cur[0]


def _unpack_row(w16):
    bf = plsc.bitcast(w16, jnp.bfloat16)
    return plsc.unpack(bf, format=plsc.PackFormat.INTERLEAVED)


def _gather_loop(idx_h, tab, idx_v, bufs, sems, wid, compute):
    pltpu.sync_copy(idx_h.at[wid], idx_v)

    def fire(j, b):
        pltpu.async_copy(tab.at[idx_v.at[j]], bufs[b], sems[b])

    for b in range(NBUF):
        fire(b, b)

    def outer(i, carry):
        for b in range(NBUF):
            j = i * NBUF + b
            pltpu.make_async_copy(tab.at[pl.ds(0, RB)], bufs[b], sems[b]).wait()
            compute(j, bufs[b])
            jn = j + NBUF

            @pl.when(jn < JBLK)
            def _():
                fire(jn, b)
        return carry

    lax.fori_loop(0, JBLK // NBUF, outer, 0)


def _ctx_body(in_idx_h, in_tab, ctx_h, in_idx_v, ctx_v, b0, b1, b2, b3, b4_, b5, b6, b7,
              s0, s1, s2, s3, s4, s5, s6, s7):
    wid = lax.axis_index("s") * NC + lax.axis_index("c")

    def compute(j, cbuf):
        for b4 in range(BPSB):
            base = b4 * CTX_LEN
            c0, c1 = _unpack_row(cbuf[base, :])
            for r in range(1, CTX_LEN):
                e, o = _unpack_row(cbuf[base + r, :])
                c0 = c0 + e
                c1 = c1 + o
            ctx_v[j, b4 * EMBED_DIM:b4 * EMBED_DIM + HALF] = c0 * _INV_CTX
            ctx_v[j, b4 * EMBED_DIM + HALF:(b4 + 1) * EMBED_DIM] = c1 * _INV_CTX

    _gather_loop(in_idx_h, in_tab, in_idx_v, (b0, b1, b2, b3, b4_, b5, b6, b7),
                 (s0, s1, s2, s3, s4, s5, s6, s7), wid, compute)
    pltpu.sync_copy(ctx_v, ctx_h.at[wid])


def _score_body(out_idx_h, out_tab, ctx_h, out_h, out_idx_v, ctx_v, score_v,
                b0, b1, b2, b3, b4_, b5, b6, b7,
                s0, s1, s2, s3, s4, s5, s6, s7):
    wid = lax.axis_index("s") * NC + lax.axis_index("c")
    pltpu.sync_copy(ctx_h.at[wid], ctx_v)

    def compute(j, obuf):
        cs = []
        for b4 in range(BPSB):
            cs.append((ctx_v[j, b4 * EMBED_DIM:b4 * EMBED_DIM + HALF],
                       ctx_v[j, b4 * EMBED_DIM + HALF:(b4 + 1) * EMBED_DIM]))
        for g in range(RB // LANES):
            ps = []
            for k in range(LANES):
                r = g * LANES + k
                c0, c1 = cs[r // OUT_LEN]
                e, o = _unpack_row(obuf[r, :])
                ps.append(e * c0 + o * c1)
            v = _reduce16(ps)
            v = 1.0 / (1.0 + jnp.exp(-v))
            score_v[j, g * LANES:(g + 1) * LANES] = v

    _gather_loop(out_idx_h, out_tab, out_idx_v, (b0, b1, b2, b3, b4_, b5, b6, b7),
                 (s0, s1, s2, s3, s4, s5, s6, s7), wid, compute)
    pltpu.sync_copy(score_v, out_h.at[wid])


_sc_params = dict(
    mesh=plsc.VectorSubcoreMesh(core_axis_name="c", subcore_axis_name="s"),
    compiler_params=pltpu.CompilerParams(use_tc_tiling_on_sc=False,
                                        needs_layout_passes=False),
)

_ring_scratch = ([pltpu.VMEM((RB, PKW), jnp.float32)] * NBUF
                 + [pltpu.SemaphoreType.DMA] * NBUF)

_cbow_ctx = functools.partial(
    pl.kernel,
    out_type=jax.ShapeDtypeStruct((NW, JBLK, CW), jnp.float32),
    scratch_types=[
        pltpu.VMEM((JBLK, RB), jnp.int32),
        pltpu.VMEM((JBLK, CW), jnp.float32),
    ] + _ring_scratch,
    **_sc_params,
)(_ctx_body)

_cbow_score = functools.partial(
    pl.kernel,
    out_type=jax.ShapeDtypeStruct((NW, JBLK, RB), jnp.float32),
    scratch_types=[
        pltpu.VMEM((JBLK, RB), jnp.int32),
        pltpu.VMEM((JBLK, CW), jnp.float32),
        pltpu.VMEM((JBLK, RB), jnp.float32),
    ] + _ring_scratch,
    **_sc_params,
)(_score_body)


_RW = 65536
_RQ = _RW // 8
_RQ_SHIFT = _RQ.bit_length() - 1


def _bf16_bits(x):
    b = lax.bitcast_convert_type(x.astype(jnp.bfloat16), jnp.uint16)
    return b.astype(jnp.int32)


def _repack_body(x_ref, o_ref):
    x = x_ref[...]
    zlo = jnp.concatenate(
        [x[0:PKW, c * _RQ:(c + 1) * _RQ] for c in range(8)], axis=0)
    zhi = jnp.concatenate(
        [x[PKW:EMBED_DIM, c * _RQ:(c + 1) * _RQ] for c in range(8)], axis=0)
    packed = (_bf16_bits(zhi.T) << 16) | _bf16_bits(zlo.T)
    o_ref[...] = lax.bitcast_convert_type(packed, jnp.float32)


def _tc_repack(tab_t):
    v = tab_t.shape[1]
    return pl.pallas_call(
        _repack_body,
        grid=(pl.cdiv(v, _RW),),
        in_specs=[pl.BlockSpec((EMBED_DIM, _RW), lambda j: (0, j))],
        out_specs=pl.BlockSpec((_RQ, 128), lambda j: (j, 0)),
        out_shape=jax.ShapeDtypeStruct((pl.cdiv(v, _RW) * _RQ, 128), jnp.float32),
    )(tab_t)


def _permute_idx(idx):
    return (idx & ~(_RW - 1)) | ((idx & (_RQ - 1)) << 3) | ((idx >> _RQ_SHIFT) & 7)


@jax.jit
def kernel(inputs, outputs, embed_inputs_table, embed_outs_table):
    in_idx = _permute_idx(inputs.astype(jnp.int32)).reshape(NW, JBLK, RB)
    out_idx = _permute_idx(outputs.astype(jnp.int32)).reshape(NW, JBLK, RB)
    t_in = _tc_repack(embed_inputs_table.T).reshape(-1, PKW)
    t_out = _tc_repack(embed_outs_table.T).reshape(-1, PKW)
    ctx = _cbow_ctx(in_idx, t_in)
    scores = _cbow_score(out_idx, t_out, ctx)
    return scores.reshape(BATCH, OUT_LEN)


# speedup vs baseline: 1.1774x; 1.1774x over previous
import functools

import jax
import jax.numpy as jnp
from jax import lax
from jax.experimental import pallas as pl
from jax.experimental.pallas import tpu as pltpu
from jax.experimental.pallas import tpu_sc as plsc

BATCH = 16384
CTX_LEN = 20
OUT_LEN = 20
EMBED_DIM = 32

NC = 2
NS = 16
NW = NC * NS
LANES = 16

BPSB = 4
RB = BPSB * CTX_LEN
JBLK = (BATCH // NW) // BPSB
NBUF = 4
HALF = EMBED_DIM // 2
CW = BPSB * EMBED_DIM
PKW = EMBED_DIM // 2

_INV_CTX = 1.0 / CTX_LEN


def _shufxor(v, s):
    perm = lax.iota(jnp.int32, LANES) ^ s
    return lax.gather(
        v,
        perm[:, None],
        dimension_numbers=lax.GatherDimensionNumbers(
            offset_dims=(), collapsed_slice_dims=(0,), start_index_map=(0,)),
        slice_sizes=(1,),
        mode=lax.GatherScatterMode.PROMISE_IN_BOUNDS,
    )


def _reduce16(ps):
    cur = ps
    for s in (8, 4, 2, 1):
        n = len(cur) // 2
        m = (lax.iota(jnp.int32, LANES) & s) == 0
        nxt = []
        for i in range(n):
            a, b = cur[i], cur[i + n]
            nxt.append(jnp.where(m, a + _shufxor(a, s), b + _shufxor(b, s)))
        cur = nxt
    return cur[0]


def _unpack_row(w16):
    bf = plsc.bitcast(w16, jnp.bfloat16)
    return plsc.unpack(bf, format=plsc.PackFormat.INTERLEAVED)


def _gather_loop(idx_h, tab, idx_v, bufs, sems, wid, compute):
    pltpu.sync_copy(idx_h.at[wid], idx_v)

    def fire(j, b):
        pltpu.async_copy(tab.at[idx_v.at[j]], bufs[b], sems[b])

    for b in range(NBUF):
        fire(b, b)

    def outer(i, carry):
        for b in range(NBUF):
            j = i * NBUF + b
            pltpu.make_async_copy(tab.at[pl.ds(0, RB)], bufs[b], sems[b]).wait()
            compute(j, bufs[b])
            jn = j + NBUF

            @pl.when(jn < JBLK)
            def _():
                fire(jn, b)
        return carry

    lax.fori_loop(0, JBLK // NBUF, outer, 0)


def _ctx_body(in_idx_h, in_tab, ctx_h, in_idx_v, ctx_v, b0, b1, b2, b3,
              s0, s1, s2, s3):
    wid = lax.axis_index("s") * NC + lax.axis_index("c")

    def compute(j, cbuf):
        for b4 in range(BPSB):
            base = b4 * CTX_LEN
            c0, c1 = _unpack_row(cbuf[base, :])
            for r in range(1, CTX_LEN):
                e, o = _unpack_row(cbuf[base + r, :])
                c0 = c0 + e
                c1 = c1 + o
            ctx_v[j, b4 * EMBED_DIM:b4 * EMBED_DIM + HALF] = c0 * _INV_CTX
            ctx_v[j, b4 * EMBED_DIM + HALF:(b4 + 1) * EMBED_DIM] = c1 * _INV_CTX

    _gather_loop(in_idx_h, in_tab, in_idx_v, (b0, b1, b2, b3),
                 (s0, s1, s2, s3), wid, compute)
    pltpu.sync_copy(ctx_v, ctx_h.at[wid])


def _score_body(out_idx_h, out_tab, ctx_h, out_h, out_idx_v, ctx_v, score_v,
                b0, b1, b2, b3, s0, s1, s2, s3):
    wid = lax.axis_index("s") * NC + lax.axis_index("c")
    pltpu.sync_copy(ctx_h.at[wid], ctx_v)

    def compute(j, obuf):
        cs = []
        for b4 in range(BPSB):
            cs.append((ctx_v[j, b4 * EMBED_DIM:b4 * EMBED_DIM + HALF],
                       ctx_v[j, b4 * EMBED_DIM + HALF:(b4 + 1) * EMBED_DIM]))
        for g in range(RB // LANES):
            ps = []
            for k in range(LANES):
                r = g * LANES + k
                c0, c1 = cs[r // OUT_LEN]
                e, o = _unpack_row(obuf[r, :])
                ps.append(e * c0 + o * c1)
            v = _reduce16(ps)
            v = 1.0 / (1.0 + jnp.exp(-v))
            score_v[j, g * LANES:(g + 1) * LANES] = v

    _gather_loop(out_idx_h, out_tab, out_idx_v, (b0, b1, b2, b3),
                 (s0, s1, s2, s3), wid, compute)
    pltpu.sync_copy(score_v, out_h.at[wid])


_sc_params = dict(
    mesh=plsc.VectorSubcoreMesh(core_axis_name="c", subcore_axis_name="s"),
    compiler_params=pltpu.CompilerParams(use_tc_tiling_on_sc=False,
                                        needs_layout_passes=False),
)

_ring_scratch = ([pltpu.VMEM((RB, PKW), jnp.float32)] * NBUF
                 + [pltpu.SemaphoreType.DMA] * NBUF)

_cbow_ctx = functools.partial(
    pl.kernel,
    out_type=jax.ShapeDtypeStruct((NW, JBLK, CW), jnp.float32),
    scratch_types=[
        pltpu.VMEM((JBLK, RB), jnp.int32),
        pltpu.VMEM((JBLK, CW), jnp.float32),
    ] + _ring_scratch,
    **_sc_params,
)(_ctx_body)

_cbow_score = functools.partial(
    pl.kernel,
    out_type=jax.ShapeDtypeStruct((NW, JBLK, RB), jnp.float32),
    scratch_types=[
        pltpu.VMEM((JBLK, RB), jnp.int32),
        pltpu.VMEM((JBLK, CW), jnp.float32),
        pltpu.VMEM((JBLK, RB), jnp.float32),
    ] + _ring_scratch,
    **_sc_params,
)(_score_body)


_RW = 65536
_RQ = _RW // 8
_RQ_SHIFT = _RQ.bit_length() - 1


def _bf16_bits(x):
    b = lax.bitcast_convert_type(x.astype(jnp.bfloat16), jnp.uint16)
    return b.astype(jnp.int32)


def _repack_body(x_ref, o_ref):
    x = x_ref[...]
    zlo = jnp.concatenate(
        [x[0:PKW, c * _RQ:(c + 1) * _RQ] for c in range(8)], axis=0)
    zhi = jnp.concatenate(
        [x[PKW:EMBED_DIM, c * _RQ:(c + 1) * _RQ] for c in range(8)], axis=0)
    packed = (_bf16_bits(zhi.T) << 16) | _bf16_bits(zlo.T)
    o_ref[...] = lax.bitcast_convert_type(packed, jnp.float32)


def _tc_repack(tab_t):
    v = tab_t.shape[1]
    return pl.pallas_call(
        _repack_body,
        grid=(pl.cdiv(v, _RW),),
        in_specs=[pl.BlockSpec((EMBED_DIM, _RW), lambda j: (0, j))],
        out_specs=pl.BlockSpec((_RQ, 128), lambda j: (j, 0)),
        out_shape=jax.ShapeDtypeStruct((pl.cdiv(v, _RW) * _RQ, 128), jnp.float32),
    )(tab_t)


def _permute_idx(idx):
    return (idx & ~(_RW - 1)) | ((idx & (_RQ - 1)) << 3) | ((idx >> _RQ_SHIFT) & 7)


@jax.jit
def kernel(inputs, outputs, embed_inputs_table, embed_outs_table):
    in_idx = _permute_idx(inputs.astype(jnp.int32)).reshape(NW, JBLK, RB)
    out_idx = _permute_idx(outputs.astype(jnp.int32)).reshape(NW, JBLK, RB)
    t_in = _tc_repack(embed_inputs_table.T).reshape(-1, PKW)
    t_out = _tc_repack(embed_outs_table.T).reshape(-1, PKW)
    ctx = _cbow_ctx(in_idx, t_in)
    scores = _cbow_score(out_idx, t_out, ctx)
    return scores.reshape(BATCH, OUT_LEN)
